# baseline (speedup 1.0000x reference)
import functools

import jax
import jax.numpy as jnp
from jax import lax
from jax.experimental import pallas as pl
from jax.experimental.pallas import tpu as pltpu
from jax.experimental.pallas import tpu_sc as plsc

NC = 2
NS = 16
NW = NC * NS
LANES = 16
CHUNK = 128


def _sc_mesh():
    return plsc.VectorSubcoreMesh(core_axis_name="c", subcore_axis_name="s")


def _make_deg_kernel(ch_per_w, n_pad):
    hrows = n_pad // 128

    @functools.partial(
        pl.kernel,
        out_type=jax.ShapeDtypeStruct((NW, hrows, 128), jnp.float32),
        mesh=_sc_mesh(),
        compiler_params=pltpu.CompilerParams(needs_layout_passes=False),
        scratch_types=[
            pltpu.VMEM((ch_per_w, CHUNK), jnp.int32),
            pltpu.VMEM((hrows, 128), jnp.float32),
        ],
    )
    def deg_kernel(row_hbm, zeros_hbm, out_hbm, idx_v, deg_v):
        c = lax.axis_index("c")
        s = lax.axis_index("s")
        wid = s * NC + c
        pltpu.sync_copy(row_hbm.at[wid], idx_v)
        pltpu.sync_copy(zeros_hbm.at[pl.ds(0, hrows)], deg_v)

        ones16 = jnp.ones((LANES,), jnp.float32)
        low_mask = jnp.full((LANES,), 127, jnp.int32)
        seven = jnp.full((LANES,), 7, jnp.int32)

        def edge_body(j, carry):
            for l in range(CHUNK // LANES):
                idx = idx_v[j, pl.ds(l * LANES, LANES)]
                hi = lax.shift_right_logical(idx, seven)
                lo = lax.bitwise_and(idx, low_mask)
                plsc.addupdate_scatter(deg_v, [hi, lo], ones16)
            return carry

        lax.fori_loop(0, ch_per_w, edge_body, 0)
        pltpu.sync_copy(deg_v, out_hbm.at[wid])

    return deg_kernel


NBUF = 2
NSEG = 2


def _make_agg_kernel(ch_per_w, n_pad, d):
    assert ch_per_w % (NSEG * NBUF) == 0
    ch_seg = ch_per_w // NSEG

    @functools.partial(
        pl.kernel,
        out_type=jax.ShapeDtypeStruct((NC, n_pad, d), jnp.float32),
        mesh=_sc_mesh(),
        compiler_params=pltpu.CompilerParams(needs_layout_passes=False),
        scratch_types=[
            pltpu.VMEM((ch_seg, CHUNK), jnp.int32),
            pltpu.VMEM((ch_seg, CHUNK), jnp.int32),
            [pltpu.VMEM((CHUNK, d), jnp.float32) for _ in range(NBUF)],
            pltpu.VMEM_SHARED((n_pad, d), jnp.float32),
            [pltpu.SemaphoreType.DMA for _ in range(NBUF)],
        ],
    )
    def agg_kernel(y_hbm, col_hbm, row_hbm, zeros_hbm, out_hbm,
                   col_v, row_v, bufs, acc_sh, sems):
        c = lax.axis_index("c")
        s = lax.axis_index("s")
        wid = s * NC + c
        rows_per_tile = n_pad // NS
        pltpu.sync_copy(
            zeros_hbm.at[pl.ds(s * rows_per_tile, rows_per_tile)],
            acc_sh.at[pl.ds(s * rows_per_tile, rows_per_tile)],
        )
        plsc.subcore_barrier()

        for seg in range(NSEG):
            pltpu.sync_copy(
                col_hbm.at[wid, pl.ds(seg * ch_seg, ch_seg)], col_v)
            pltpu.sync_copy(
                row_hbm.at[wid, pl.ds(seg * ch_seg, ch_seg)], row_v)
            for b in range(NBUF):
                pltpu.async_copy(y_hbm.at[col_v.at[b]], bufs[b], sems[b])

            def group_body(g, carry):
                base = g * NBUF
                for b in range(NBUF):
                    j = base + b
                    pltpu.make_async_copy(
                        y_hbm.at[col_v.at[j]], bufs[b], sems[b]).wait()
                    pltpu.sync_copy(
                        bufs[b], acc_sh.at[row_v.at[j]], add=True)
                    nxt = j + NBUF

                    @pl.when(nxt < ch_seg)
                    def _():
                        pltpu.async_copy(
                            y_hbm.at[col_v.at[nxt]], bufs[b], sems[b])
                return carry

            lax.fori_loop(0, ch_seg // NBUF, group_body, 0)
        plsc.subcore_barrier()
        pltpu.sync_copy(
            acc_sh.at[pl.ds(s * rows_per_tile, rows_per_tile)],
            out_hbm.at[c, pl.ds(s * rows_per_tile, rows_per_tile)],
        )

    return agg_kernel


def _mm_body(x_ref, w_ref, z_ref):
    z_ref[...] = lax.dot_general(
        x_ref[...], w_ref[...],
        (((1,), (1,)), ((), ())),
        preferred_element_type=jnp.float32,
    )


def _scale_body(degp_ref, z_ref, y_ref, dis_ref):
    deg = jnp.sum(degp_ref[...], axis=0).reshape(-1)
    dis = jnp.where(deg > 0.0, lax.rsqrt(deg), 0.0)
    dis_ref[...] = dis
    n = z_ref.shape[0]
    n_pad = y_ref.shape[0]
    y_ref[pl.ds(0, n), :] = dis[:n, None] * z_ref[...]
    y_ref[pl.ds(n, n_pad - n), :] = jnp.zeros(
        (n_pad - n, z_ref.shape[1]), jnp.float32)


def _fin_body(s_ref, dis_ref, b_ref, o_ref):
    n = o_ref.shape[0]
    ssum = s_ref[0, pl.ds(0, n), :] + s_ref[1, pl.ds(0, n), :]
    val = dis_ref[...][:n, None] * ssum + b_ref[...]
    o_ref[...] = jnp.maximum(val, 0.0)


def kernel(x, edge_index, W, b):
    n, d_in = x.shape
    d_out = W.shape[0]
    e = edge_index.shape[1]

    ch_per_w = -(-e // (NW * CHUNK))
    ch_per_w = -(-ch_per_w // (NSEG * NBUF)) * (NSEG * NBUF)
    e_pad = NW * ch_per_w * CHUNK
    n_pad = -(-n // (NS * LANES)) * (NS * LANES)

    row = edge_index[0]
    col = edge_index[1]
    epw = -(-e // NW)
    epw_pad = ch_per_w * CHUNK

    gpad = NW * epw - e
    karr = jnp.arange(gpad, dtype=jnp.int32)
    row_deg_g = jnp.concatenate([row, n + karr % (n_pad - n)])
    row_agg_g = jnp.concatenate([row, (karr * 37) % n_pad])
    col_g = jnp.concatenate([col, n + (karr * 3) % (n_pad - n)])

    wpad = epw_pad - epw
    w_ids = jnp.arange(NW, dtype=jnp.int32)[:, None]
    k2 = jnp.arange(wpad, dtype=jnp.int32)[None, :]
    row_deg_p = jnp.concatenate(
        [row_deg_g.reshape(NW, epw),
         n + (w_ids * 7 + k2) % (n_pad - n)], axis=1)
    row_agg_p = jnp.concatenate(
        [row_agg_g.reshape(NW, epw),
         (w_ids * 613 + k2 * 37) % n_pad], axis=1)
    col_p = jnp.concatenate(
        [col_g.reshape(NW, epw),
         n + (w_ids * 11 + k2 * 3) % (n_pad - n)], axis=1)
    row_deg_p = row_deg_p.reshape(NW, ch_per_w, CHUNK)
    row_agg_p = row_agg_p.reshape(NW, ch_per_w, CHUNK)
    col_p = col_p.reshape(NW, ch_per_w, CHUNK)

    zeros_nd = jnp.zeros((n_pad, d_in), jnp.float32)
    deg_parts = _make_deg_kernel(ch_per_w, n_pad)(row_deg_p, zeros_nd)

    z = pl.pallas_call(
        _mm_body,
        out_shape=jax.ShapeDtypeStruct((n, d_in), jnp.float32),
    )(x, W)

    y, dis = pl.pallas_call(
        _scale_body,
        out_shape=(
            jax.ShapeDtypeStruct((n_pad, d_in), jnp.float32),
            jax.ShapeDtypeStruct((n_pad,), jnp.float32),
        ),
    )(deg_parts, z)

    s_parts = _make_agg_kernel(ch_per_w, n_pad, d_in)(
        y, col_p, row_agg_p, zeros_nd)

    out = pl.pallas_call(
        _fin_body,
        out_shape=jax.ShapeDtypeStruct((n, d_out), jnp.float32),
    )(s_parts, dis, b.reshape(1, d_out))
    return out


# speedup vs baseline: 1.0223x; 1.0223x over previous
import functools

import jax
import jax.numpy as jnp
from jax import lax
from jax.experimental import pallas as pl
from jax.experimental.pallas import tpu as pltpu
from jax.experimental.pallas import tpu_sc as plsc

NC = 2
NS = 16
NW = NC * NS
LANES = 16
CHUNK = 128


def _sc_mesh():
    return plsc.VectorSubcoreMesh(core_axis_name="c", subcore_axis_name="s")


def _make_deg_kernel(e, epw, n_pad):
    epw_al = -(-epw // LANES) * LANES + 2 * LANES

    @functools.partial(
        pl.kernel,
        out_type=jax.ShapeDtypeStruct((NW, n_pad), jnp.float32),
        mesh=_sc_mesh(),
        compiler_params=pltpu.CompilerParams(needs_layout_passes=False),
        scratch_types=[
            pltpu.VMEM((epw_al,), jnp.int32),
            pltpu.VMEM((n_pad,), jnp.float32),
        ],
    )
    def deg_kernel(row_hbm, out_hbm, idx_v, deg_v):
        c = lax.axis_index("c")
        s = lax.axis_index("s")
        wid = s * NC + c
        lo = wid * epw
        hi = jnp.minimum(lo + epw, e)
        base = (jnp.maximum(jnp.minimum(lo, e - epw_al), 0) // 8) * 8
        pltpu.sync_copy(row_hbm.at[pl.ds(base, epw_al)], idx_v)

        zeros16 = jnp.zeros((LANES,), jnp.float32)

        def zero_body(i, carry):
            deg_v[pl.ds(i * LANES, LANES)] = zeros16
            return carry

        lax.fori_loop(0, n_pad // LANES, zero_body, 0, unroll=8)

        ones16 = jnp.ones((LANES,), jnp.float32)
        lane = lax.iota(jnp.int32, LANES)

        def edge_body(j, carry):
            idx = idx_v[pl.ds(j * LANES, LANES)]
            g = base + j * LANES + lane
            msk = jnp.logical_and(g >= lo, g < hi)
            plsc.addupdate_scatter(deg_v, [idx], ones16, mask=msk)
            return carry

        lax.fori_loop(0, epw_al // LANES, edge_body, 0, unroll=8)
        pltpu.sync_copy(deg_v, out_hbm.at[wid])

    return deg_kernel


NBUF = 2
NSEG = 2


def _make_agg_kernel(ch_per_w, n_pad, d):
    assert ch_per_w % (NSEG * NBUF) == 0
    ch_seg = ch_per_w // NSEG

    @functools.partial(
        pl.kernel,
        out_type=jax.ShapeDtypeStruct((NC, n_pad, d), jnp.float32),
        mesh=_sc_mesh(),
        compiler_params=pltpu.CompilerParams(needs_layout_passes=False),
        scratch_types=[
            pltpu.VMEM((ch_seg, CHUNK), jnp.int32),
            pltpu.VMEM((ch_seg, CHUNK), jnp.int32),
            [pltpu.VMEM((CHUNK, d), jnp.float32) for _ in range(NBUF)],
            pltpu.VMEM_SHARED((n_pad, d), jnp.float32),
            [pltpu.SemaphoreType.DMA for _ in range(NBUF)],
        ],
    )
    def agg_kernel(y_hbm, col_hbm, row_hbm, zeros_hbm, out_hbm,
                   col_v, row_v, bufs, acc_sh, sems):
        c = lax.axis_index("c")
        s = lax.axis_index("s")
        wid = s * NC + c
        rows_per_tile = n_pad // NS
        pltpu.sync_copy(
            zeros_hbm.at[pl.ds(s * rows_per_tile, rows_per_tile)],
            acc_sh.at[pl.ds(s * rows_per_tile, rows_per_tile)],
        )
        plsc.subcore_barrier()

        for seg in range(NSEG):
            pltpu.sync_copy(
                col_hbm.at[wid, pl.ds(seg * ch_seg, ch_seg)], col_v)
            pltpu.sync_copy(
                row_hbm.at[wid, pl.ds(seg * ch_seg, ch_seg)], row_v)
            for b in range(NBUF):
                pltpu.async_copy(y_hbm.at[col_v.at[b]], bufs[b], sems[b])

            def group_body(g, carry):
                base = g * NBUF
                for b in range(NBUF):
                    j = base + b
                    pltpu.make_async_copy(
                        y_hbm.at[col_v.at[j]], bufs[b], sems[b]).wait()
                    pltpu.sync_copy(
                        bufs[b], acc_sh.at[row_v.at[j]], add=True)
                    nxt = j + NBUF

                    @pl.when(nxt < ch_seg)
                    def _():
                        pltpu.async_copy(
                            y_hbm.at[col_v.at[nxt]], bufs[b], sems[b])
                return carry

            lax.fori_loop(0, ch_seg // NBUF, group_body, 0)
        plsc.subcore_barrier()
        pltpu.sync_copy(
            acc_sh.at[pl.ds(s * rows_per_tile, rows_per_tile)],
            out_hbm.at[c, pl.ds(s * rows_per_tile, rows_per_tile)],
        )

    return agg_kernel


def _prep_body(degp_ref, x_ref, w_ref, y_ref, dis_ref):
    deg = jnp.sum(degp_ref[...], axis=0)
    dis = jnp.where(deg > 0.0, lax.rsqrt(deg), 0.0)
    dis_ref[...] = dis
    n = x_ref.shape[0]
    n_pad = y_ref.shape[0]
    z = lax.dot_general(
        x_ref[...], w_ref[...],
        (((1,), (1,)), ((), ())),
        preferred_element_type=jnp.float32,
    )
    y_ref[pl.ds(0, n), :] = dis[:n, None] * z
    y_ref[pl.ds(n, n_pad - n), :] = jnp.zeros(
        (n_pad - n, z.shape[1]), jnp.float32)


def _fin_body(s_ref, dis_ref, b_ref, o_ref):
    n = o_ref.shape[0]
    ssum = s_ref[0, pl.ds(0, n), :] + s_ref[1, pl.ds(0, n), :]
    val = dis_ref[...][:n, None] * ssum + b_ref[...]
    o_ref[...] = jnp.maximum(val, 0.0)


def kernel(x, edge_index, W, b):
    n, d_in = x.shape
    d_out = W.shape[0]
    e = edge_index.shape[1]

    ch_per_w = -(-e // (NW * CHUNK))
    ch_per_w = -(-ch_per_w // (NSEG * NBUF)) * (NSEG * NBUF)
    e_pad = NW * ch_per_w * CHUNK
    n_pad = -(-n // (NS * LANES)) * (NS * LANES)

    row = edge_index[0]
    col = edge_index[1]
    epw = -(-e // NW)
    epw_pad = ch_per_w * CHUNK

    gpad = NW * epw - e
    karr = jnp.arange(gpad, dtype=jnp.int32)
    row_agg_g = jnp.concatenate([row, (karr * 37) % n_pad])
    col_g = jnp.concatenate([col, n + (karr * 3) % (n_pad - n)])

    wpad = epw_pad - epw
    w_ids = jnp.arange(NW, dtype=jnp.int32)[:, None]
    k2 = jnp.arange(wpad, dtype=jnp.int32)[None, :]
    row_agg_p = jnp.concatenate(
        [row_agg_g.reshape(NW, epw),
         (w_ids * 613 + k2 * 37) % n_pad], axis=1)
    col_p = jnp.concatenate(
        [col_g.reshape(NW, epw),
         n + (w_ids * 11 + k2 * 3) % (n_pad - n)], axis=1)
    row_agg_p = row_agg_p.reshape(NW, ch_per_w, CHUNK)
    col_p = col_p.reshape(NW, ch_per_w, CHUNK)

    zeros_nd = jnp.zeros((n_pad, d_in), jnp.float32)
    deg_parts = _make_deg_kernel(e, epw, n_pad)(row)

    y, dis = pl.pallas_call(
        _prep_body,
        out_shape=(
            jax.ShapeDtypeStruct((n_pad, d_in), jnp.float32),
            jax.ShapeDtypeStruct((n_pad,), jnp.float32),
        ),
    )(deg_parts, x, W)

    s_parts = _make_agg_kernel(ch_per_w, n_pad, d_in)(
        y, col_p, row_agg_p, zeros_nd)

    out = pl.pallas_call(
        _fin_body,
        out_shape=jax.ShapeDtypeStruct((n, d_out), jnp.float32),
    )(s_parts, dis, b.reshape(1, d_out))
    return out


# speedup vs baseline: 1.0458x; 1.0230x over previous
import functools

import jax
import jax.numpy as jnp
from jax import lax
from jax.experimental import pallas as pl
from jax.experimental.pallas import tpu as pltpu
from jax.experimental.pallas import tpu_sc as plsc

NC = 2
NS = 16
NW = NC * NS
LANES = 16
CHUNK = 128


def _sc_mesh():
    return plsc.VectorSubcoreMesh(core_axis_name="c", subcore_axis_name="s")


def _make_deg_kernel(e, epw, n_pad):
    epw_al = -(-epw // LANES) * LANES + 2 * LANES

    @functools.partial(
        pl.kernel,
        out_type=jax.ShapeDtypeStruct((NW, n_pad), jnp.float32),
        mesh=_sc_mesh(),
        compiler_params=pltpu.CompilerParams(needs_layout_passes=False),
        scratch_types=[
            pltpu.VMEM((epw_al,), jnp.int32),
            pltpu.VMEM((n_pad,), jnp.float32),
        ],
    )
    def deg_kernel(row_hbm, out_hbm, idx_v, deg_v):
        c = lax.axis_index("c")
        s = lax.axis_index("s")
        wid = s * NC + c
        lo = wid * epw
        hi = jnp.minimum(lo + epw, e)
        base = (jnp.maximum(jnp.minimum(lo, e - epw_al), 0) // 8) * 8
        pltpu.sync_copy(row_hbm.at[pl.ds(base, epw_al)], idx_v)

        zeros16 = jnp.zeros((LANES,), jnp.float32)

        def zero_body(i, carry):
            deg_v[pl.ds(i * LANES, LANES)] = zeros16
            return carry

        lax.fori_loop(0, n_pad // LANES, zero_body, 0, unroll=8)

        ones16 = jnp.ones((LANES,), jnp.float32)
        lane = lax.iota(jnp.int32, LANES)

        def edge_body(j, carry):
            idx = idx_v[pl.ds(j * LANES, LANES)]
            g = base + j * LANES + lane
            msk = jnp.logical_and(g >= lo, g < hi)
            plsc.addupdate_scatter(deg_v, [idx], ones16, mask=msk)
            return carry

        lax.fori_loop(0, epw_al // LANES, edge_body, 0, unroll=8)
        pltpu.sync_copy(deg_v, out_hbm.at[wid])

    return deg_kernel


NBUF = 2
NSEG = 2


def _make_agg_kernel(ch_per_w, n_pad, d):
    assert ch_per_w % (NSEG * NBUF) == 0
    ch_seg = ch_per_w // NSEG

    @functools.partial(
        pl.kernel,
        out_type=jax.ShapeDtypeStruct((NC, n_pad, d), jnp.float32),
        mesh=_sc_mesh(),
        compiler_params=pltpu.CompilerParams(needs_layout_passes=False),
        scratch_types=[
            pltpu.VMEM((ch_seg, CHUNK), jnp.int32),
            pltpu.VMEM((ch_seg, CHUNK), jnp.int32),
            [pltpu.VMEM((CHUNK, d), jnp.float32) for _ in range(NBUF)],
            pltpu.VMEM_SHARED((n_pad, d), jnp.float32),
            [pltpu.SemaphoreType.DMA for _ in range(NBUF)],
            pltpu.SemaphoreType.DMA,
        ],
    )
    def agg_kernel(y_hbm, col_hbm, row_hbm, zeros_hbm, out_hbm,
                   col_v, row_v, bufs, acc_sh, sems, zsem):
        c = lax.axis_index("c")
        s = lax.axis_index("s")
        wid = s * NC + c
        rows_per_tile = n_pad // NS
        zcp = pltpu.async_copy(
            zeros_hbm.at[pl.ds(s * rows_per_tile, rows_per_tile)],
            acc_sh.at[pl.ds(s * rows_per_tile, rows_per_tile)],
            zsem,
        )
        first = True
        for seg in range(NSEG):
            pltpu.sync_copy(
                col_hbm.at[wid, pl.ds(seg * ch_seg, ch_seg)], col_v)
            pltpu.sync_copy(
                row_hbm.at[wid, pl.ds(seg * ch_seg, ch_seg)], row_v)
            for b in range(NBUF):
                pltpu.async_copy(y_hbm.at[col_v.at[b]], bufs[b], sems[b])
            if first:
                first = False
                zcp.wait()
                plsc.subcore_barrier()

            def group_body(g, carry):
                base = g * NBUF
                for b in range(NBUF):
                    j = base + b
                    pltpu.make_async_copy(
                        y_hbm.at[col_v.at[j]], bufs[b], sems[b]).wait()
                    pltpu.sync_copy(
                        bufs[b], acc_sh.at[row_v.at[j]], add=True)
                    nxt = j + NBUF

                    @pl.when(nxt < ch_seg)
                    def _():
                        pltpu.async_copy(
                            y_hbm.at[col_v.at[nxt]], bufs[b], sems[b])
                return carry

            lax.fori_loop(0, ch_seg // NBUF, group_body, 0)
        plsc.subcore_barrier()
        pltpu.sync_copy(
            acc_sh.at[pl.ds(s * rows_per_tile, rows_per_tile)],
            out_hbm.at[c, pl.ds(s * rows_per_tile, rows_per_tile)],
        )

    return agg_kernel


def _prep_body(degp_ref, x_ref, w_ref, y_ref, dis_ref):
    deg = jnp.sum(degp_ref[...], axis=0)
    dis = jnp.where(deg > 0.0, lax.rsqrt(deg), 0.0)
    dis_ref[...] = dis
    n = x_ref.shape[0]
    n_pad = y_ref.shape[0]
    z = lax.dot_general(
        x_ref[...], w_ref[...],
        (((1,), (1,)), ((), ())),
        preferred_element_type=jnp.float32,
    )
    y_ref[pl.ds(0, n), :] = dis[:n, None] * z
    y_ref[pl.ds(n, n_pad - n), :] = jnp.zeros(
        (n_pad - n, z.shape[1]), jnp.float32)


def _fin_body(s_ref, dis_ref, b_ref, o_ref):
    n = o_ref.shape[0]
    ssum = s_ref[0, pl.ds(0, n), :] + s_ref[1, pl.ds(0, n), :]
    val = dis_ref[...][:n, None] * ssum + b_ref[...]
    o_ref[...] = jnp.maximum(val, 0.0)


def kernel(x, edge_index, W, b):
    n, d_in = x.shape
    d_out = W.shape[0]
    e = edge_index.shape[1]

    ch_per_w = -(-e // (NW * CHUNK))
    ch_per_w = -(-ch_per_w // (NSEG * NBUF)) * (NSEG * NBUF)
    e_pad = NW * ch_per_w * CHUNK
    n_pad = -(-n // (NS * LANES)) * (NS * LANES)

    row = edge_index[0]
    col = edge_index[1]
    epw = -(-e // NW)
    epw_pad = ch_per_w * CHUNK

    gpad = NW * epw - e
    karr = jnp.arange(gpad, dtype=jnp.int32)
    row_agg_g = jnp.concatenate([row, (karr * 37) % n_pad])
    col_g = jnp.concatenate([col, n + (karr * 3) % (n_pad - n)])
    wpad = epw_pad - epw
    w_ids = jnp.arange(NW, dtype=jnp.int32)[:, None]
    k2 = jnp.arange(wpad, dtype=jnp.int32)[None, :]
    row_agg_p = jnp.concatenate(
        [row_agg_g.reshape(NW, epw),
         (w_ids * 613 + k2 * 37) % n_pad], axis=1)
    col_p = jnp.concatenate(
        [col_g.reshape(NW, epw),
         n + (w_ids * 11 + k2 * 3) % (n_pad - n)], axis=1)
    row_agg_p = row_agg_p.reshape(NW, ch_per_w, CHUNK)
    col_p = col_p.reshape(NW, ch_per_w, CHUNK)

    zeros_nd = jnp.zeros((n_pad, d_in), jnp.float32)
    deg_parts = _make_deg_kernel(e, epw, n_pad)(row)

    y, dis = pl.pallas_call(
        _prep_body,
        out_shape=(
            jax.ShapeDtypeStruct((n_pad, d_in), jnp.float32),
            jax.ShapeDtypeStruct((n_pad,), jnp.float32),
        ),
    )(deg_parts, x, W)

    s_parts = _make_agg_kernel(ch_per_w, n_pad, d_in)(
        y, col_p, row_agg_p, zeros_nd)

    out = pl.pallas_call(
        _fin_body,
        out_shape=jax.ShapeDtypeStruct((n, d_out), jnp.float32),
    )(s_parts, dis, b.reshape(1, d_out))
    return out


# speedup vs baseline: 1.0966x; 1.0485x over previous
import functools

import jax
import jax.numpy as jnp
from jax import lax
from jax.experimental import pallas as pl
from jax.experimental.pallas import tpu as pltpu
from jax.experimental.pallas import tpu_sc as plsc

NC = 2
NS = 16
NW = NC * NS
LANES = 16
CHUNK = 128


def _sc_mesh():
    return plsc.VectorSubcoreMesh(core_axis_name="c", subcore_axis_name="s")


def _make_deg_kernel(e, ch_per_w, n_pad):
    epw_pad = ch_per_w * CHUNK

    @functools.partial(
        pl.kernel,
        out_type=jax.ShapeDtypeStruct((NW, n_pad), jnp.float32),
        mesh=_sc_mesh(),
        compiler_params=pltpu.CompilerParams(needs_layout_passes=False),
        scratch_types=[
            pltpu.VMEM((ch_per_w, CHUNK), jnp.int32),
            pltpu.VMEM((n_pad,), jnp.float32),
        ],
    )
    def deg_kernel(edges_hbm, out_hbm, idx_v, deg_v):
        c = lax.axis_index("c")
        s = lax.axis_index("s")
        wid = s * NC + c
        pltpu.sync_copy(edges_hbm.at[0, wid], idx_v)

        zeros16 = jnp.zeros((LANES,), jnp.float32)

        def zero_body(i, carry):
            deg_v[pl.ds(i * LANES, LANES)] = zeros16
            return carry

        lax.fori_loop(0, n_pad // LANES, zero_body, 0, unroll=8)

        ones16 = jnp.ones((LANES,), jnp.float32)
        lane = lax.iota(jnp.int32, LANES)
        base = wid * epw_pad

        def edge_body(p, carry):
            j = p // (CHUNK // LANES)
            l = p % (CHUNK // LANES)
            idx = idx_v[j, pl.ds(l * LANES, LANES)]
            msk = base + p * LANES + lane < e
            plsc.addupdate_scatter(deg_v, [idx], ones16, mask=msk)
            return carry

        lax.fori_loop(0, epw_pad // LANES, edge_body, 0, unroll=8)
        pltpu.sync_copy(deg_v, out_hbm.at[wid])

    return deg_kernel


NBUF = 2
NSEG = 2


def _make_agg_kernel(ch_per_w, n_pad, d):
    assert ch_per_w % (NSEG * NBUF) == 0
    ch_seg = ch_per_w // NSEG

    @functools.partial(
        pl.kernel,
        out_type=jax.ShapeDtypeStruct((NC, n_pad, d), jnp.float32),
        mesh=_sc_mesh(),
        compiler_params=pltpu.CompilerParams(needs_layout_passes=False),
        scratch_types=[
            pltpu.VMEM((ch_seg, CHUNK), jnp.int32),
            pltpu.VMEM((ch_seg, CHUNK), jnp.int32),
            [pltpu.VMEM((CHUNK, d), jnp.float32) for _ in range(NBUF)],
            pltpu.VMEM_SHARED((n_pad, d), jnp.float32),
            [pltpu.SemaphoreType.DMA for _ in range(NBUF)],
            pltpu.SemaphoreType.DMA,
        ],
    )
    def agg_kernel(y_hbm, edges_hbm, zeros_hbm, out_hbm,
                   col_v, row_v, bufs, acc_sh, sems, zsem):
        c = lax.axis_index("c")
        s = lax.axis_index("s")
        wid = s * NC + c
        rows_per_tile = n_pad // NS
        zcp = pltpu.async_copy(
            zeros_hbm.at[pl.ds(s * rows_per_tile, rows_per_tile)],
            acc_sh.at[pl.ds(s * rows_per_tile, rows_per_tile)],
            zsem,
        )
        first = True
        for seg in range(NSEG):
            pltpu.sync_copy(
                edges_hbm.at[1, wid, pl.ds(seg * ch_seg, ch_seg)], col_v)
            pltpu.sync_copy(
                edges_hbm.at[0, wid, pl.ds(seg * ch_seg, ch_seg)], row_v)
            for b in range(NBUF):
                pltpu.async_copy(y_hbm.at[col_v.at[b]], bufs[b], sems[b])
            if first:
                first = False
                zcp.wait()
                plsc.subcore_barrier()

            def group_body(g, carry):
                base = g * NBUF
                for b in range(NBUF):
                    j = base + b
                    pltpu.make_async_copy(
                        y_hbm.at[col_v.at[j]], bufs[b], sems[b]).wait()
                    pltpu.sync_copy(
                        bufs[b], acc_sh.at[row_v.at[j]], add=True)
                    nxt = j + NBUF

                    @pl.when(nxt < ch_seg)
                    def _():
                        pltpu.async_copy(
                            y_hbm.at[col_v.at[nxt]], bufs[b], sems[b])
                return carry

            lax.fori_loop(0, ch_seg // NBUF, group_body, 0)
        plsc.subcore_barrier()
        pltpu.sync_copy(
            acc_sh.at[pl.ds(s * rows_per_tile, rows_per_tile)],
            out_hbm.at[c, pl.ds(s * rows_per_tile, rows_per_tile)],
        )

    return agg_kernel


def _prep_body(degp_ref, x_ref, w_ref, y_ref, dis_ref):
    deg = jnp.sum(degp_ref[...], axis=0)
    dis = jnp.where(deg > 0.0, lax.rsqrt(deg), 0.0)
    dis_ref[...] = dis
    n = x_ref.shape[0]
    n_pad = y_ref.shape[0]
    z = lax.dot_general(
        x_ref[...], w_ref[...],
        (((1,), (1,)), ((), ())),
        preferred_element_type=jnp.float32,
    )
    y_ref[pl.ds(0, n), :] = dis[:n, None] * z
    y_ref[pl.ds(n, n_pad - n), :] = jnp.zeros(
        (n_pad - n, z.shape[1]), jnp.float32)


def _fin_body(s_ref, dis_ref, b_ref, o_ref):
    n = o_ref.shape[0]
    ssum = s_ref[0, pl.ds(0, n), :] + s_ref[1, pl.ds(0, n), :]
    val = dis_ref[...][:n, None] * ssum + b_ref[...]
    o_ref[...] = jnp.maximum(val, 0.0)


def kernel(x, edge_index, W, b):
    n, d_in = x.shape
    d_out = W.shape[0]
    e = edge_index.shape[1]

    ch_per_w = -(-e // (NW * CHUNK))
    ch_per_w = -(-ch_per_w // (NSEG * NBUF)) * (NSEG * NBUF)
    e_pad = NW * ch_per_w * CHUNK
    n_pad = -(-n // (NS * LANES)) * (NS * LANES)

    karr = jnp.arange(e_pad - e, dtype=jnp.int32)
    pad2 = jnp.stack([
        (karr * 37) % n_pad,
        n + (karr * 3) % (n_pad - n),
    ])
    edges4 = jnp.concatenate([edge_index, pad2], axis=1).reshape(
        2, NW, ch_per_w, CHUNK)

    zeros_nd = jnp.zeros((n_pad, d_in), jnp.float32)
    deg_parts = _make_deg_kernel(e, ch_per_w, n_pad)(edges4)

    y, dis = pl.pallas_call(
        _prep_body,
        out_shape=(
            jax.ShapeDtypeStruct((n_pad, d_in), jnp.float32),
            jax.ShapeDtypeStruct((n_pad,), jnp.float32),
        ),
    )(deg_parts, x, W)

    s_parts = _make_agg_kernel(ch_per_w, n_pad, d_in)(
        y, edges4, zeros_nd)

    out = pl.pallas_call(
        _fin_body,
        out_shape=jax.ShapeDtypeStruct((n, d_out), jnp.float32),
    )(s_parts, dis, b.reshape(1, d_out))
    return out
